# baseline (speedup 1.0000x reference)
import jax
import jax.numpy as jnp
from jax.experimental import pallas as pl
from jax.experimental.pallas import tpu as pltpu

N = 10000
K = 16
D_LOCAL = 128
D_PAIR = 16
A = 20
BN = 200
SLOTS = 4
STEPS = N // BN


def _potts_block(local_ref, pair_ref, nbr_ref, w1_ref, w2_ref, wp1_ref,
                 wp2_ref, bias_ref, pssm_ref, contact_hbm, scratch_ref,
                 sems_ref):
    i = pl.program_id(0)
    slot = jax.lax.rem(i, SLOTS)

    @pl.when(i >= SLOTS)
    def _wait_prev():
        prev = i - SLOTS
        pltpu.make_async_copy(
            scratch_ref.at[slot],
            contact_hbm.at[pl.ds(prev * BN, BN)],
            sems_ref.at[slot],
        ).wait()

    h = jax.nn.silu(jnp.dot(local_ref[...], w1_ref[...],
                            preferred_element_type=jnp.float32))
    pssm_ref[...] = jnp.dot(h, w2_ref[...],
                            preferred_element_type=jnp.float32) + bias_ref[...]

    x = pair_ref[...].reshape(BN * K, D_PAIR)
    hp = jax.nn.silu(jnp.dot(x, wp1_ref[...],
                             preferred_element_type=jnp.float32))
    y = jnp.dot(hp, wp2_ref[...], preferred_element_type=jnp.float32)

    nbr = nbr_ref[...]
    node_ids = i * BN + jax.lax.broadcasted_iota(jnp.int32, (BN, K), 0)
    m = ((nbr != node_ids) & (nbr != -1)).astype(jnp.float32)
    scratch_ref[slot] = y.reshape(BN, K, A * A) * m[:, :, None]

    pltpu.make_async_copy(
        scratch_ref.at[slot],
        contact_hbm.at[pl.ds(i * BN, BN)],
        sems_ref.at[slot],
    ).start()

    @pl.when(i == STEPS - 1)
    def _drain():
        for back in range(SLOTS - 1, -1, -1):
            step = STEPS - 1 - back
            s = step % SLOTS
            pltpu.make_async_copy(
                scratch_ref.at[s],
                contact_hbm.at[pl.ds(step * BN, BN)],
                sems_ref.at[s],
            ).wait()


@jax.jit
def kernel(local, pair, extra_pair, neighbours, extra_pair_mask, mask,
           W1, W2, Wp1, Wp2, aa_bias):
    del extra_pair, extra_pair_mask, mask
    bias2d = aa_bias.reshape(1, A)
    pssm, contact = pl.pallas_call(
        _potts_block,
        grid=(STEPS,),
        in_specs=[
            pl.BlockSpec((BN, D_LOCAL), lambda i: (i, 0)),
            pl.BlockSpec((BN, K, D_PAIR), lambda i: (i, 0, 0)),
            pl.BlockSpec((BN, K), lambda i: (i, 0)),
            pl.BlockSpec((D_LOCAL, 2 * D_LOCAL), lambda i: (0, 0)),
            pl.BlockSpec((2 * D_LOCAL, A), lambda i: (0, 0)),
            pl.BlockSpec((D_PAIR, 2 * D_PAIR), lambda i: (0, 0)),
            pl.BlockSpec((2 * D_PAIR, A * A), lambda i: (0, 0)),
            pl.BlockSpec((1, A), lambda i: (0, 0)),
        ],
        out_specs=[
            pl.BlockSpec((BN, A), lambda i: (i, 0)),
            pl.BlockSpec(memory_space=pl.ANY),
        ],
        out_shape=[
            jax.ShapeDtypeStruct((N, A), jnp.float32),
            jax.ShapeDtypeStruct((N, K, A * A), jnp.float32),
        ],
        scratch_shapes=[
            pltpu.VMEM((SLOTS, BN, K, A * A), jnp.float32),
            pltpu.SemaphoreType.DMA((SLOTS,)),
        ],
    )(local, pair, neighbours, W1, W2, Wp1, Wp2, bias2d)
    return pssm, contact.reshape(N, K, A, A)
